# baseline (speedup 1.0000x reference)
import jax
import jax.numpy as jnp
from jax.experimental import pallas as pl
from jax.experimental.pallas import tpu as pltpu

DIM = 2048
BR = 512
R = DIM // BR
NLAYERS = 6
TOTAL = NLAYERS * R
NBUF = 8
LOOKAHEAD = NBUF - 1


def _celu(x):
    return jnp.where(x > 0, x, jnp.exp(jnp.minimum(x, 0.0)) - 1.0)


HALF = BR // 2


def _issue(w_refs, wbuf, sem, i2, r2, slot2):
    for c in range(NLAYERS):
        @pl.when(i2 == c)
        def _(c=c):
            pltpu.make_async_copy(
                w_refs[c].at[pl.ds(r2 * BR, HALF), :],
                wbuf.at[slot2, pl.ds(0, HALF)],
                sem.at[slot2, 0],
            ).start()
            pltpu.make_async_copy(
                w_refs[c].at[pl.ds(r2 * BR + HALF, HALF), :],
                wbuf.at[slot2, pl.ds(HALF, HALF)],
                sem.at[slot2, 1],
            ).start()


def _mlp_kernel(zt_ref, b_ref,
                w0, w1, w2, w3, w4, w5,
                out_ref, wbuf, h, acc, sem):
    w_refs = (w0, w1, w2, w3, w4, w5)
    k = pl.program_id(0)
    i = k // R
    r = k % R
    slot = jax.lax.rem(k, NBUF)
    par = jax.lax.rem(i, 2)

    @pl.when(k == 0)
    def _():
        h[0, :, :] = zt_ref[...]
        for k2 in range(LOOKAHEAD):
            _issue(w_refs, wbuf, sem, k2 // R, k2 % R, k2 % NBUF)

    k2 = k + LOOKAHEAD

    @pl.when(k2 < TOTAL)
    def _():
        _issue(w_refs, wbuf, sem, k2 // R, jax.lax.rem(k2, R),
               jax.lax.rem(k2, NBUF))

    for c in range(NLAYERS):
        @pl.when(i == c)
        def _(c=c):
            pltpu.make_async_copy(
                w_refs[c].at[pl.ds(r * BR, HALF), :],
                wbuf.at[slot, pl.ds(0, HALF)],
                sem.at[slot, 0],
            ).wait()
            pltpu.make_async_copy(
                w_refs[c].at[pl.ds(r * BR + HALF, HALF), :],
                wbuf.at[slot, pl.ds(HALF, HALF)],
                sem.at[slot, 1],
            ).wait()

    x = h[pl.ds(par, 1), pl.ds(r * BR, BR), :].reshape(BR, 1)
    w = wbuf[pl.ds(slot, 1)].reshape(BR, DIM)
    part = jnp.sum(x * w, axis=0, keepdims=True)

    @pl.when(r == 0)
    def _():
        acc[...] = part

    @pl.when(r > 0)
    def _():
        acc[...] = acc[...] + part

    @pl.when(r == R - 1)
    def _():
        y = acc[...] + b_ref[pl.ds(i, 1), :]

        @pl.when(i < NLAYERS - 1)
        def _():
            h[pl.ds(1 - par, 1), :, :] = jnp.transpose(_celu(y))[None]

        @pl.when(i == NLAYERS - 1)
        def _():
            out_ref[...] = y


def kernel(z, cur_W0, cur_b0, cur_W1, cur_b1, cur_W2, cur_b2,
           cur_W3, cur_b3, cur_W4, cur_b4, out_W, out_b):
    b_all = jnp.stack([cur_b0, cur_b1, cur_b2, cur_b3, cur_b4, out_b])

    in_specs = [
        pl.BlockSpec((DIM, 1), lambda k: (0, 0)),
        pl.BlockSpec((NLAYERS, DIM), lambda k: (0, 0)),
    ] + [pl.BlockSpec(memory_space=pl.ANY)] * NLAYERS

    out = pl.pallas_call(
        _mlp_kernel,
        grid=(TOTAL,),
        in_specs=in_specs,
        out_specs=pl.BlockSpec((1, DIM), lambda k: (0, 0)),
        out_shape=jax.ShapeDtypeStruct((1, DIM), jnp.float32),
        scratch_shapes=[
            pltpu.VMEM((NBUF, BR, DIM), jnp.float32),
            pltpu.VMEM((2, DIM, 1), jnp.float32),
            pltpu.VMEM((1, DIM), jnp.float32),
            pltpu.SemaphoreType.DMA((NBUF, 2)),
        ],
        compiler_params=pltpu.CompilerParams(
            dimension_semantics=("arbitrary",)),
    )(z.reshape(DIM, 1), b_all,
      cur_W0, cur_W1, cur_W2, cur_W3, cur_W4, out_W)
    return out


# speedup vs baseline: 1.0258x; 1.0258x over previous
import jax
import jax.numpy as jnp
from jax.experimental import pallas as pl
from jax.experimental.pallas import tpu as pltpu

DIM = 2048
BR = 1024
R = DIM // BR
NLAYERS = 6
TOTAL = NLAYERS * R
NBUF = 4
LOOKAHEAD = NBUF - 1


def _celu(x):
    return jnp.where(x > 0, x, jnp.exp(jnp.minimum(x, 0.0)) - 1.0)


HALF = BR // 2


def _issue(w_refs, wbuf, sem, i2, r2, slot2):
    for c in range(NLAYERS):
        @pl.when(i2 == c)
        def _(c=c):
            pltpu.make_async_copy(
                w_refs[c].at[pl.ds(r2 * BR, HALF), :],
                wbuf.at[slot2, pl.ds(0, HALF)],
                sem.at[slot2, 0],
            ).start()
            pltpu.make_async_copy(
                w_refs[c].at[pl.ds(r2 * BR + HALF, HALF), :],
                wbuf.at[slot2, pl.ds(HALF, HALF)],
                sem.at[slot2, 1],
            ).start()


def _mlp_kernel(zt_ref, b_ref,
                w0, w1, w2, w3, w4, w5,
                out_ref, wbuf, h, acc, sem):
    w_refs = (w0, w1, w2, w3, w4, w5)
    k = pl.program_id(0)
    i = k // R
    r = k % R
    slot = jax.lax.rem(k, NBUF)
    par = jax.lax.rem(i, 2)

    @pl.when(k == 0)
    def _():
        h[0, :, :] = zt_ref[...]
        for k2 in range(LOOKAHEAD):
            _issue(w_refs, wbuf, sem, k2 // R, k2 % R, k2 % NBUF)

    k2 = k + LOOKAHEAD

    @pl.when(k2 < TOTAL)
    def _():
        _issue(w_refs, wbuf, sem, k2 // R, jax.lax.rem(k2, R),
               jax.lax.rem(k2, NBUF))

    for c in range(NLAYERS):
        @pl.when(i == c)
        def _(c=c):
            pltpu.make_async_copy(
                w_refs[c].at[pl.ds(r * BR, HALF), :],
                wbuf.at[slot, pl.ds(0, HALF)],
                sem.at[slot, 0],
            ).wait()
            pltpu.make_async_copy(
                w_refs[c].at[pl.ds(r * BR + HALF, HALF), :],
                wbuf.at[slot, pl.ds(HALF, HALF)],
                sem.at[slot, 1],
            ).wait()

    x = h[pl.ds(par, 1), pl.ds(r * BR, BR), :].reshape(BR, 1)
    w = wbuf[pl.ds(slot, 1)].reshape(BR, DIM)
    part = jnp.sum(x * w, axis=0, keepdims=True)

    @pl.when(r == 0)
    def _():
        acc[...] = part

    @pl.when(r > 0)
    def _():
        acc[...] = acc[...] + part

    @pl.when(r == R - 1)
    def _():
        y = acc[...] + b_ref[pl.ds(i, 1), :]

        @pl.when(i < NLAYERS - 1)
        def _():
            h[pl.ds(1 - par, 1), :, :] = jnp.transpose(_celu(y))[None]

        @pl.when(i == NLAYERS - 1)
        def _():
            out_ref[...] = y


def kernel(z, cur_W0, cur_b0, cur_W1, cur_b1, cur_W2, cur_b2,
           cur_W3, cur_b3, cur_W4, cur_b4, out_W, out_b):
    b_all = jnp.stack([cur_b0, cur_b1, cur_b2, cur_b3, cur_b4, out_b])

    in_specs = [
        pl.BlockSpec((DIM, 1), lambda k: (0, 0)),
        pl.BlockSpec((NLAYERS, DIM), lambda k: (0, 0)),
    ] + [pl.BlockSpec(memory_space=pl.ANY)] * NLAYERS

    out = pl.pallas_call(
        _mlp_kernel,
        grid=(TOTAL,),
        in_specs=in_specs,
        out_specs=pl.BlockSpec((1, DIM), lambda k: (0, 0)),
        out_shape=jax.ShapeDtypeStruct((1, DIM), jnp.float32),
        scratch_shapes=[
            pltpu.VMEM((NBUF, BR, DIM), jnp.float32),
            pltpu.VMEM((2, DIM, 1), jnp.float32),
            pltpu.VMEM((1, DIM), jnp.float32),
            pltpu.SemaphoreType.DMA((NBUF, 2)),
        ],
        compiler_params=pltpu.CompilerParams(
            dimension_semantics=("arbitrary",)),
    )(z.reshape(DIM, 1), b_all,
      cur_W0, cur_W1, cur_W2, cur_W3, cur_W4, out_W)
    return out


# speedup vs baseline: 1.0326x; 1.0066x over previous
import jax
import jax.numpy as jnp
from jax.experimental import pallas as pl
from jax.experimental.pallas import tpu as pltpu

DIM = 2048
BR = 512
R = DIM // BR
NLAYERS = 6
TOTAL = NLAYERS * R
NBUF = 6
LOOKAHEAD = NBUF - 1


def _celu(x):
    return jnp.where(x > 0, x, jnp.exp(jnp.minimum(x, 0.0)) - 1.0)


HALF = BR // 2


def _issue(w_refs, wbuf, sem, i2, r2, slot2):
    for c in range(NLAYERS):
        @pl.when(i2 == c)
        def _(c=c):
            pltpu.make_async_copy(
                w_refs[c].at[pl.ds(r2 * BR, HALF), :],
                wbuf.at[slot2, pl.ds(0, HALF)],
                sem.at[slot2, 0],
            ).start()
            pltpu.make_async_copy(
                w_refs[c].at[pl.ds(r2 * BR + HALF, HALF), :],
                wbuf.at[slot2, pl.ds(HALF, HALF)],
                sem.at[slot2, 1],
            ).start()


def _mlp_kernel(zt_ref, b_ref,
                w0, w1, w2, w3, w4, w5,
                out_ref, wbuf, h, acc, sem):
    w_refs = (w0, w1, w2, w3, w4, w5)
    k = pl.program_id(0)
    i = k // R
    r = k % R
    slot = jax.lax.rem(k, NBUF)
    par = jax.lax.rem(i, 2)

    @pl.when(k == 0)
    def _():
        h[0, :, :] = zt_ref[...]
        for k2 in range(LOOKAHEAD):
            _issue(w_refs, wbuf, sem, k2 // R, k2 % R, k2 % NBUF)

    k2 = k + LOOKAHEAD

    @pl.when(k2 < TOTAL)
    def _():
        _issue(w_refs, wbuf, sem, k2 // R, jax.lax.rem(k2, R),
               jax.lax.rem(k2, NBUF))

    for c in range(NLAYERS):
        @pl.when(i == c)
        def _(c=c):
            pltpu.make_async_copy(
                w_refs[c].at[pl.ds(r * BR, HALF), :],
                wbuf.at[slot, pl.ds(0, HALF)],
                sem.at[slot, 0],
            ).wait()
            pltpu.make_async_copy(
                w_refs[c].at[pl.ds(r * BR + HALF, HALF), :],
                wbuf.at[slot, pl.ds(HALF, HALF)],
                sem.at[slot, 1],
            ).wait()

    x = h[pl.ds(par, 1), pl.ds(r * BR, BR), :].reshape(BR, 1)
    w = wbuf[pl.ds(slot, 1)].reshape(BR, DIM)
    part = jnp.sum(x * w, axis=0, keepdims=True)

    @pl.when(r == 0)
    def _():
        acc[...] = part

    @pl.when(r > 0)
    def _():
        acc[...] = acc[...] + part

    @pl.when(r == R - 1)
    def _():
        y = acc[...] + b_ref[pl.ds(i, 1), :]

        @pl.when(i < NLAYERS - 1)
        def _():
            h[pl.ds(1 - par, 1), :, :] = jnp.transpose(_celu(y))[None]

        @pl.when(i == NLAYERS - 1)
        def _():
            out_ref[...] = y


def kernel(z, cur_W0, cur_b0, cur_W1, cur_b1, cur_W2, cur_b2,
           cur_W3, cur_b3, cur_W4, cur_b4, out_W, out_b):
    b_all = jnp.stack([cur_b0, cur_b1, cur_b2, cur_b3, cur_b4, out_b])

    in_specs = [
        pl.BlockSpec((DIM, 1), lambda k: (0, 0)),
        pl.BlockSpec((NLAYERS, DIM), lambda k: (0, 0)),
    ] + [pl.BlockSpec(memory_space=pl.ANY)] * NLAYERS

    out = pl.pallas_call(
        _mlp_kernel,
        grid=(TOTAL,),
        in_specs=in_specs,
        out_specs=pl.BlockSpec((1, DIM), lambda k: (0, 0)),
        out_shape=jax.ShapeDtypeStruct((1, DIM), jnp.float32),
        scratch_shapes=[
            pltpu.VMEM((NBUF, BR, DIM), jnp.float32),
            pltpu.VMEM((2, DIM, 1), jnp.float32),
            pltpu.VMEM((1, DIM), jnp.float32),
            pltpu.SemaphoreType.DMA((NBUF, 2)),
        ],
        compiler_params=pltpu.CompilerParams(
            dimension_semantics=("arbitrary",)),
    )(z.reshape(DIM, 1), b_all,
      cur_W0, cur_W1, cur_W2, cur_W3, cur_W4, out_W)
    return out
